# baseline (speedup 1.0000x reference)
import functools

import jax
import jax.numpy as jnp
from jax.experimental import pallas as pl
from jax.experimental.pallas import tpu as pltpu

B = 4096
D = 512
R = 256
NBLK = B // R
PAD = 128

_K1 = (1832780943, 270669613)
_K2 = (64467757, 2916123636)
_TINY = float(jnp.finfo(jnp.float32).tiny)
_ROTS = ((13, 15, 26, 6), (17, 29, 16, 24))


def _gumbel_bits(key, n):
    k0 = jnp.uint32(key[0])
    k1 = jnp.uint32(key[1])
    ks = (k0, k1, k0 ^ k1 ^ jnp.uint32(0x1BD11BDA))
    x0 = jnp.broadcast_to(ks[0], n.shape) + jnp.uint32(0)
    x1 = n + ks[1]

    def rounds(x0, x1, rs):
        for r in rs:
            x0 = x0 + x1
            x1 = (jax.lax.shift_left(x1, jnp.uint32(r))
                  | jax.lax.shift_right_logical(x1, jnp.uint32(32 - r)))
            x1 = x1 ^ x0
        return x0, x1

    for i in range(5):
        x0, x1 = rounds(x0, x1, _ROTS[i % 2])
        x0 = x0 + ks[(i + 1) % 3]
        x1 = x1 + ks[(i + 2) % 3] + jnp.uint32(i + 1)
    bits = x0 ^ x1

    fb = jax.lax.shift_right_logical(bits, jnp.uint32(9)) | jnp.uint32(0x3F800000)
    f = jax.lax.bitcast_convert_type(fb, jnp.float32) - 1.0
    u = jnp.maximum(jnp.float32(_TINY), f * 1.0 + jnp.float32(_TINY))
    return -jnp.log(-jnp.log(u))


def _sample_project_body(li_ref, lt_ref, ai_ref, at_ref,
                         pwi_ref, pwt_ref,
                         idxt_ref, idxi_ref, pi_ref, pt_ref):
    i = pl.program_id(0)
    r0 = i * R

    col = jax.lax.broadcasted_iota(jnp.int32, (R, B), 1)
    row = r0 + jax.lax.broadcasted_iota(jnp.int32, (R, B), 0)
    diag = col == row
    flat = (row * B + col).astype(jnp.uint32)

    def draw(x, key):
        g = _gumbel_bits(key, flat)
        m = jnp.max(x, axis=1, keepdims=True)
        u = jnp.exp(x - m)
        s = jnp.sum(u, axis=1, keepdims=True)
        w = u / s
        w = jnp.where(diag, 0.0, w)
        v = jnp.where(w > 0, jnp.log(w), -jnp.inf) + g
        vmax = jnp.max(v, axis=1, keepdims=True)
        cand = jnp.where(v == vmax, col, B)
        return jnp.min(cand, axis=1).astype(jnp.int32)

    idxt_ref[0, pl.ds(r0, R)] = draw(li_ref[...], _K1)
    idxi_ref[0, pl.ds(r0, R)] = draw(lt_ref[...], _K2)

    pi_ref[...] = jnp.dot(ai_ref[...], pwi_ref[...],
                          preferred_element_type=jnp.float32)
    pt_ref[...] = jnp.dot(at_ref[...], pwt_ref[...],
                          preferred_element_type=jnp.float32)


def _finalize_body(idxt_ref, idxi_ref, pi_ref, pt_ref, pb_ref,
                   lg0_ref, lg1_ref, lg2_ref, loss_ref):
    i = pl.program_id(0)
    r0 = i * R

    idx_t = idxt_ref[0, pl.ds(r0, R)]
    idx_i = idxi_ref[0, pl.ds(r0, R)]
    col = jax.lax.broadcasted_iota(jnp.int32, (R, B), 1)
    oh_t = (col == idx_t[:, None]).astype(jnp.float32)
    oh_i = (col == idx_i[:, None]).astype(jnp.float32)
    gath_t = jnp.dot(oh_t, pt_ref[...], preferred_element_type=jnp.float32)
    gath_i = jnp.dot(oh_i, pi_ref[...], preferred_element_type=jnp.float32)

    pi_blk = pi_ref[pl.ds(r0, R), :]
    pt_blk = pt_ref[pl.ds(r0, R), :]
    pb = pb_ref[...]

    lg0 = pi_blk + pt_blk + pb
    lg1 = pi_blk + gath_t + pb
    lg2 = gath_i + pt_blk + pb
    lg0_ref[...] = lg0
    lg1_ref[...] = lg1
    lg2_ref[...] = lg2

    def logp(lg, want_pos):
        a = lg[:, 0:1]
        b = lg[:, 1:2]
        mx = jnp.maximum(a, b)
        lse = jnp.log(jnp.exp(a - mx) + jnp.exp(b - mx))
        sel = b if want_pos else a
        return (sel - mx) - lse

    partial = (jnp.sum(logp(lg0, True)) + jnp.sum(logp(lg1, False))
               + jnp.sum(logp(lg2, False)))

    @pl.when(i == 0)
    def _():
        loss_ref[...] = jnp.zeros_like(loss_ref)

    loss_ref[...] += jnp.full((1, 1), partial, jnp.float32)

    @pl.when(i == NBLK - 1)
    def _():
        loss_ref[...] = loss_ref[...] * (-1.0 / (3.0 * B))


@functools.partial(jax.jit, static_argnames=())
def kernel(all_image_features, all_text_features, logits_per_image,
           logits_per_text, proj_w, proj_b):
    pw_img = jnp.zeros((D, PAD), jnp.float32).at[:, :2].set(proj_w[:D])
    pw_txt = jnp.zeros((D, PAD), jnp.float32).at[:, :2].set(proj_w[D:])
    pb_pad = jnp.zeros((1, PAD), jnp.float32).at[0, :2].set(proj_b)

    row_spec = pl.BlockSpec((R, B), lambda i: (i, 0))
    feat_spec = pl.BlockSpec((R, D), lambda i: (i, 0))
    full_w = pl.BlockSpec((D, PAD), lambda i: (0, 0))
    idx_spec = pl.BlockSpec((1, B), lambda i: (0, 0))
    proj_out = pl.BlockSpec((R, PAD), lambda i: (i, 0))

    idxt, idxi, pi, pt = pl.pallas_call(
        _sample_project_body,
        grid=(NBLK,),
        in_specs=[row_spec, row_spec,
                  feat_spec, feat_spec, full_w, full_w],
        out_specs=[idx_spec, idx_spec, proj_out, proj_out],
        out_shape=[
            jax.ShapeDtypeStruct((1, B), jnp.int32),
            jax.ShapeDtypeStruct((1, B), jnp.int32),
            jax.ShapeDtypeStruct((B, PAD), jnp.float32),
            jax.ShapeDtypeStruct((B, PAD), jnp.float32),
        ],
    )(logits_per_image, logits_per_text,
      all_image_features, all_text_features, pw_img, pw_txt)

    full_proj = pl.BlockSpec((B, PAD), lambda i: (0, 0))
    pb_spec = pl.BlockSpec((1, PAD), lambda i: (0, 0))
    lg_spec = pl.BlockSpec((R, PAD), lambda i: (i, 0))
    loss_spec = pl.BlockSpec((1, 1), lambda i: (0, 0))

    lg0, lg1, lg2, loss = pl.pallas_call(
        _finalize_body,
        grid=(NBLK,),
        in_specs=[idx_spec, idx_spec, full_proj, full_proj, pb_spec],
        out_specs=[lg_spec, lg_spec, lg_spec, loss_spec],
        out_shape=[
            jax.ShapeDtypeStruct((B, PAD), jnp.float32),
            jax.ShapeDtypeStruct((B, PAD), jnp.float32),
            jax.ShapeDtypeStruct((B, PAD), jnp.float32),
            jax.ShapeDtypeStruct((1, 1), jnp.float32),
        ],
    )(idxt, idxi, pi, pt, pb_pad)

    logits = jnp.concatenate([lg0[:, :2], lg1[:, :2], lg2[:, :2]], axis=0)
    itm_labels = jnp.concatenate([
        jnp.ones((B,), dtype=jnp.int32),
        jnp.zeros((B,), dtype=jnp.int32),
        jnp.zeros((B,), dtype=jnp.int32),
    ])
    return loss[0, 0], logits, itm_labels


# speedup vs baseline: 1.3510x; 1.3510x over previous
import functools

import jax
import jax.numpy as jnp
from jax.experimental import pallas as pl
from jax.experimental.pallas import tpu as pltpu

B = 4096
D = 512
R = 256
NBLK = B // R
PAD = 128

_K1 = (1832780943, 270669613)
_K2 = (64467757, 2916123636)
_TINY = float(jnp.finfo(jnp.float32).tiny)
_ROTS = ((13, 15, 26, 6), (17, 29, 16, 24))


def _gumbel_from_bits(bits):
    fb = jax.lax.shift_right_logical(bits, jnp.uint32(9)) | jnp.uint32(0x3F800000)
    f = jax.lax.bitcast_convert_type(fb, jnp.float32) - 1.0
    u = jnp.maximum(jnp.float32(_TINY), f * 1.0 + jnp.float32(_TINY))
    return -jnp.log(-jnp.log(u))


def _sample_project_body(li_ref, b1_ref, lt_ref, b2_ref, ai_ref, at_ref,
                         pwi_ref, pwt_ref,
                         idxt_ref, idxi_ref, pi_ref, pt_ref):
    i = pl.program_id(0)
    r0 = i * R

    col = jax.lax.broadcasted_iota(jnp.int32, (R, B), 1)
    row = r0 + jax.lax.broadcasted_iota(jnp.int32, (R, B), 0)
    diag = col == row

    def draw(x, bits):
        g = _gumbel_from_bits(bits)
        m = jnp.max(x, axis=1, keepdims=True)
        u = jnp.exp(x - m)
        s = jnp.sum(u, axis=1, keepdims=True)
        w = u / s
        w = jnp.where(diag, 0.0, w)
        v = jnp.where(w > 0, jnp.log(w), -jnp.inf) + g
        vmax = jnp.max(v, axis=1, keepdims=True)
        cand = jnp.where(v == vmax, col, B)
        return jnp.min(cand, axis=1).astype(jnp.int32)

    idxt_ref[0, pl.ds(r0, R)] = draw(li_ref[...], b1_ref[...])
    idxi_ref[0, pl.ds(r0, R)] = draw(lt_ref[...], b2_ref[...])

    pi_ref[...] = jnp.dot(ai_ref[...], pwi_ref[...],
                          preferred_element_type=jnp.float32)
    pt_ref[...] = jnp.dot(at_ref[...], pwt_ref[...],
                          preferred_element_type=jnp.float32)


def _finalize_body(idxt_ref, idxi_ref, pi_ref, pt_ref, pb_ref,
                   lg0_ref, lg1_ref, lg2_ref, loss_ref):
    i = pl.program_id(0)
    r0 = i * R

    idx_t = idxt_ref[0, pl.ds(r0, R)]
    idx_i = idxi_ref[0, pl.ds(r0, R)]
    col = jax.lax.broadcasted_iota(jnp.int32, (R, B), 1)
    oh_t = (col == idx_t[:, None]).astype(jnp.float32)
    oh_i = (col == idx_i[:, None]).astype(jnp.float32)
    gath_t = jnp.dot(oh_t, pt_ref[...], preferred_element_type=jnp.float32)
    gath_i = jnp.dot(oh_i, pi_ref[...], preferred_element_type=jnp.float32)

    pi_blk = pi_ref[pl.ds(r0, R), :]
    pt_blk = pt_ref[pl.ds(r0, R), :]
    pb = pb_ref[...]

    lg0 = pi_blk + pt_blk + pb
    lg1 = pi_blk + gath_t + pb
    lg2 = gath_i + pt_blk + pb
    lg0_ref[...] = lg0
    lg1_ref[...] = lg1
    lg2_ref[...] = lg2

    def logp(lg, want_pos):
        a = lg[:, 0:1]
        b = lg[:, 1:2]
        mx = jnp.maximum(a, b)
        lse = jnp.log(jnp.exp(a - mx) + jnp.exp(b - mx))
        sel = b if want_pos else a
        return (sel - mx) - lse

    partial = (jnp.sum(logp(lg0, True)) + jnp.sum(logp(lg1, False))
               + jnp.sum(logp(lg2, False)))

    @pl.when(i == 0)
    def _():
        loss_ref[...] = jnp.zeros_like(loss_ref)

    loss_ref[...] += jnp.full((1, 1), partial, jnp.float32)

    @pl.when(i == NBLK - 1)
    def _():
        loss_ref[...] = loss_ref[...] * (-1.0 / (3.0 * B))


@functools.partial(jax.jit, static_argnames=())
def kernel(all_image_features, all_text_features, logits_per_image,
           logits_per_text, proj_w, proj_b):
    skey = jax.random.key(42)
    k1, k2 = jax.random.split(skey)
    bits1 = jax.random.bits(k1, (B, B), jnp.uint32)
    bits2 = jax.random.bits(k2, (B, B), jnp.uint32)

    pw_img = jnp.zeros((D, PAD), jnp.float32).at[:, :2].set(proj_w[:D])
    pw_txt = jnp.zeros((D, PAD), jnp.float32).at[:, :2].set(proj_w[D:])
    pb_pad = jnp.zeros((1, PAD), jnp.float32).at[0, :2].set(proj_b)

    row_spec = pl.BlockSpec((R, B), lambda i: (i, 0))
    feat_spec = pl.BlockSpec((R, D), lambda i: (i, 0))
    full_w = pl.BlockSpec((D, PAD), lambda i: (0, 0))
    idx_spec = pl.BlockSpec((1, B), lambda i: (0, 0))
    proj_out = pl.BlockSpec((R, PAD), lambda i: (i, 0))

    idxt, idxi, pi, pt = pl.pallas_call(
        _sample_project_body,
        grid=(NBLK,),
        in_specs=[row_spec, row_spec, row_spec, row_spec,
                  feat_spec, feat_spec, full_w, full_w],
        out_specs=[idx_spec, idx_spec, proj_out, proj_out],
        out_shape=[
            jax.ShapeDtypeStruct((1, B), jnp.int32),
            jax.ShapeDtypeStruct((1, B), jnp.int32),
            jax.ShapeDtypeStruct((B, PAD), jnp.float32),
            jax.ShapeDtypeStruct((B, PAD), jnp.float32),
        ],
    )(logits_per_image, bits1, logits_per_text, bits2,
      all_image_features, all_text_features, pw_img, pw_txt)

    full_proj = pl.BlockSpec((B, PAD), lambda i: (0, 0))
    pb_spec = pl.BlockSpec((1, PAD), lambda i: (0, 0))
    lg_spec = pl.BlockSpec((R, PAD), lambda i: (i, 0))
    loss_spec = pl.BlockSpec((1, 1), lambda i: (0, 0))

    lg0, lg1, lg2, loss = pl.pallas_call(
        _finalize_body,
        grid=(NBLK,),
        in_specs=[idx_spec, idx_spec, full_proj, full_proj, pb_spec],
        out_specs=[lg_spec, lg_spec, lg_spec, loss_spec],
        out_shape=[
            jax.ShapeDtypeStruct((B, PAD), jnp.float32),
            jax.ShapeDtypeStruct((B, PAD), jnp.float32),
            jax.ShapeDtypeStruct((B, PAD), jnp.float32),
            jax.ShapeDtypeStruct((1, 1), jnp.float32),
        ],
    )(idxt, idxi, pi, pt, pb_pad)

    logits = jnp.concatenate([lg0[:, :2], lg1[:, :2], lg2[:, :2]], axis=0)
    itm_labels = jnp.concatenate([
        jnp.ones((B,), dtype=jnp.int32),
        jnp.zeros((B,), dtype=jnp.int32),
        jnp.zeros((B,), dtype=jnp.int32),
    ])
    return loss[0, 0], logits, itm_labels
